# baseline (speedup 1.0000x reference)
import functools

import numpy as np
import jax
import jax.numpy as jnp
from jax import lax
from jax.experimental import pallas as pl
from jax.experimental.pallas import tpu as pltpu
from jax.experimental.pallas import tpu_sc as plsc

N = 10000
E = 320000
D = 128
NB = 8
CUTOFF = 5.0
GAMMA = 10.0
NORM_FACTOR = 32.0

NC = 2
NS = 16
NW = NC * NS
EPT = E // NW
K = 64
NCH = 156
KT = EPT - NCH * K
NPAIR = NCH // 2
NP = 10112
RPT = NP // NS

_CENTERS = [float(c) for c in np.linspace(0.0, CUTOFF, NB)]


_RW = 128
_RROWS = E // _RW
_BR = 20


def _pack2(x):
    u = lax.bitcast_convert_type(x, jnp.uint32) + jnp.uint32(0x8000)
    lo = lax.shift_right_logical(u[:, : D // 2], jnp.uint32(16))
    hi = u[:, D // 2 :] & jnp.uint32(0xFFFF0000)
    return lax.bitcast_convert_type(lo | hi, jnp.int32)


def _radial_body(rij_ref, wr_ref, ef_ref, out_ref, eo_ref):
    eo_ref[...] = ef_ref[...]
    r = rij_ref[0]
    fc = 0.5 * (jnp.cos(np.pi / CUTOFF * r) + 1.0)
    fc = jnp.where(r < CUTOFF, fc, 0.0)
    lhs = jnp.concatenate(
        [jnp.exp(-GAMMA * (r - _CENTERS[b]) ** 2) * fc for b in range(NB)],
        axis=0)
    out_ref[...] = _pack2(lax.dot_general(
        lhs, wr_ref[...], (((0,), (0,)), ((), ())),
        preferred_element_type=jnp.float32))


def _radial(rij, W_radial, edge_feat):
    return pl.pallas_call(
        _radial_body,
        grid=(_RROWS // _BR,),
        in_specs=[
            pl.BlockSpec((1, 1, _BR * _RW), lambda i: (i, 0, 0)),
            pl.BlockSpec((NB, D), lambda i: (0, 0)),
            pl.BlockSpec((_BR * _RW, D), lambda i: (i, 0)),
        ],
        out_specs=[
            pl.BlockSpec((_BR * _RW, D // 2), lambda i: (i, 0)),
            pl.BlockSpec((_BR * _RW, D), lambda i: (i, 0)),
        ],
        out_shape=[
            jax.ShapeDtypeStruct((E, D // 2), jnp.int32),
            jax.ShapeDtypeStruct((E, D), jnp.float32),
        ],
    )(rij.reshape(_RROWS // _BR, 1, _BR * _RW), W_radial, edge_feat)


_MESH = plsc.VectorSubcoreMesh(core_axis_name="c", subcore_axis_name="s")


@functools.partial(
    pl.kernel,
    out_type=jax.ShapeDtypeStruct((NC, NP, D), jnp.float32),
    mesh=_MESH,
    compiler_params=pltpu.CompilerParams(needs_layout_passes=False),
    scratch_types=[
        pltpu.VMEM((8 * K,), jnp.int32),
        pltpu.VMEM((8 * K,), jnp.int32),
        pltpu.VMEM((2, K, D // 2), jnp.int32),
        pltpu.VMEM((2, K, D), jnp.float32),
        pltpu.VMEM((2, K, D), jnp.float32),
        pltpu.VMEM_SHARED((NP, D), jnp.float32),
        pltpu.SemaphoreType.DMA,
        pltpu.SemaphoreType.DMA,
        pltpu.SemaphoreType.DMA,
        pltpu.SemaphoreType.DMA,
        pltpu.SemaphoreType.DMA,
        pltpu.SemaphoreType.DMA,
        pltpu.SemaphoreType.DMA,
        pltpu.SemaphoreType.DMA,
        pltpu.SemaphoreType.DMA,
        pltpu.SemaphoreType.DMA,
    ],
)
def _edge_agg(node_hbm, r_hbm, idxi_hbm, idxj_hbm, out_hbm,
              ij_v, ii_v, r_v, g_v, msg_v, agg,
              semr0, semr1, semg0, semg1, sems0, sems1,
              semij0, semij1, semii0, semii1):
    cid = lax.axis_index("c")
    sid = lax.axis_index("s")
    wid = cid * NS + sid
    ebase = wid * EPT
    semr = (semr0, semr1)
    semg = (semg0, semg1)
    sems = (sems0, sems1)
    semij = (semij0, semij1)
    semii = (semii0, semii1)

    def _zrow(k, _):
        for v in range(D // 16):
            msg_v[0, k, pl.ds(v * 16, 16)] = jnp.zeros((16,), jnp.float32)
        return 0

    lax.fori_loop(0, K, _zrow, 0)
    for z in range(RPT // K):
        pltpu.sync_copy(msg_v.at[0], agg.at[pl.ds(sid * RPT + z * K, K), :])
    _zrem = RPT - (RPT // K) * K
    if _zrem:
        pltpu.sync_copy(
            msg_v.at[0, pl.ds(0, _zrem), :],
            agg.at[pl.ds(sid * RPT + (RPT // K) * K, _zrem), :])
    plsc.subcore_barrier()

    def _issue_idx(p, off4, sem_par):
        pltpu.async_copy(idxj_hbm.at[pl.ds(ebase + p * 2 * K, 2 * K)],
                         ij_v.at[pl.ds(off4, 2 * K)], semij[sem_par])
        pltpu.async_copy(idxi_hbm.at[pl.ds(ebase + p * 2 * K, 2 * K)],
                         ii_v.at[pl.ds(off4, 2 * K)], semii[sem_par])

    def _wait_idx(p, off4, sem_par):
        pltpu.make_async_copy(idxj_hbm.at[pl.ds(ebase + p * 2 * K, 2 * K)],
                              ij_v.at[pl.ds(off4, 2 * K)],
                              semij[sem_par]).wait()
        pltpu.make_async_copy(idxi_hbm.at[pl.ds(ebase + p * 2 * K, 2 * K)],
                              ii_v.at[pl.ds(off4, 2 * K)],
                              semii[sem_par]).wait()

    def _issue_data(t, off4, c, slot):
        pltpu.async_copy(r_hbm.at[pl.ds(ebase + t * K, K), :],
                         r_v.at[slot], semr[slot])
        pltpu.async_copy(node_hbm.at[ij_v.at[pl.ds(off4 + c * K, K)]],
                         g_v.at[slot], semg[slot])

    def _mul(slot, rows):
        def _row(k, _):
            for v in range(D // 32):
                sl = pl.ds(v * 16, 16)
                sh = pl.ds(D // 2 + v * 16, 16)
                rw = r_v[slot, k, sl]
                ra = plsc.bitcast(lax.shift_left(rw, 16), jnp.float32)
                rb = plsc.bitcast(rw & jnp.int32(-65536), jnp.float32)
                msg_v[slot, k, sl] = ra * g_v[slot, k, sl]
                msg_v[slot, k, sh] = rb * g_v[slot, k, sh]
            return 0

        lax.fori_loop(0, rows, _row, 0)

    def _process(t, off4, c, slot, first):
        pltpu.make_async_copy(r_hbm.at[pl.ds(ebase + t * K, K), :],
                              r_v.at[slot], semr[slot]).wait()
        pltpu.make_async_copy(node_hbm.at[ij_v.at[pl.ds(off4 + c * K, K)]],
                              g_v.at[slot], semg[slot]).wait()

        @pl.when(jnp.logical_not(first))
        def _():
            pltpu.make_async_copy(
                msg_v.at[slot], agg.at[ii_v.at[pl.ds(off4 + c * K, K)]],
                sems[slot]).wait()

        _mul(slot, K)
        pltpu.async_copy(msg_v.at[slot],
                         agg.at[ii_v.at[pl.ds(off4 + c * K, K)]],
                         sems[slot], add=True)

    pltpu.sync_copy(idxj_hbm.at[pl.ds(ebase, 2 * K)],
                    ij_v.at[pl.ds(0, 2 * K)])
    pltpu.sync_copy(idxi_hbm.at[pl.ds(ebase, 2 * K)],
                    ii_v.at[pl.ds(0, 2 * K)])
    _issue_idx(1, 2 * K, 1)
    _issue_data(0, 0, 0, 0)
    _issue_data(1, 0, 1, 1)

    def _pair_body(u, par):
        off4 = (u % 4) * 2 * K
        off4n = ((u + 1) % 4) * 2 * K

        @pl.when(u + 2 < NPAIR)
        def _():
            _issue_idx(u + 2, ((u + 2) % 4) * 2 * K, par)

        _process(2 * u, off4, 0, 0, u == 0)

        @pl.when(u + 1 < NPAIR)
        def _():
            _wait_idx(u + 1, off4n, 1 - par)
            _issue_data(2 * u + 2, off4n, 0, 0)

        _process(2 * u + 1, off4, 1, 1, u == 0)

        @pl.when(u + 1 < NPAIR)
        def _():
            _issue_data(2 * u + 3, off4n, 1, 1)

    def _pairs(h, _):
        _pair_body(2 * h, 0)
        _pair_body(2 * h + 1, 1)
        return 0

    lax.fori_loop(0, NPAIR // 2, _pairs, 0)

    for slot in range(2):
        pltpu.make_async_copy(
            msg_v.at[slot], agg.at[ii_v.at[pl.ds(slot * K, K)]],
            sems[slot]).wait()

    tbase = ebase + NCH * K
    pltpu.sync_copy(idxj_hbm.at[pl.ds(tbase, KT)], ij_v.at[pl.ds(0, KT)])
    pltpu.sync_copy(idxi_hbm.at[pl.ds(tbase, KT)], ii_v.at[pl.ds(0, KT)])
    pltpu.sync_copy(r_hbm.at[pl.ds(tbase, KT), :], r_v.at[0, pl.ds(0, KT), :])
    pltpu.async_copy(node_hbm.at[ij_v.at[pl.ds(0, KT)]],
                     g_v.at[0, pl.ds(0, KT), :], semg[0]).wait()
    _mul(0, KT)
    pltpu.sync_copy(msg_v.at[0, pl.ds(0, KT), :],
                    agg.at[ii_v.at[pl.ds(0, KT)]], add=True)
    plsc.subcore_barrier()

    pltpu.sync_copy(agg.at[pl.ds(sid * RPT, RPT), :],
                    out_hbm.at[cid, pl.ds(sid * RPT, RPT), :])


_BN = 1000


def _update_body(p_ref, nf_ref, ws_ref, b_ref, o_ref):
    agg = p_ref[0] + p_ref[1]
    h = jnp.dot(agg, ws_ref[...], preferred_element_type=jnp.float32)
    h = h + b_ref[...]
    o_ref[...] = nf_ref[...] + h * jax.nn.sigmoid(h)


def _update(partials, node_feat, W_self, b_self):
    return pl.pallas_call(
        _update_body,
        grid=(N // _BN,),
        in_specs=[
            pl.BlockSpec((NC, _BN, D), lambda i: (0, i, 0)),
            pl.BlockSpec((_BN, D), lambda i: (i, 0)),
            pl.BlockSpec((D, D), lambda i: (0, 0)),
            pl.BlockSpec((1, D), lambda i: (0, 0)),
        ],
        out_specs=pl.BlockSpec((_BN, D), lambda i: (i, 0)),
        out_shape=jax.ShapeDtypeStruct((N, D), jnp.float32),
    )(partials, node_feat, W_self, b_self.reshape(1, D))


def kernel(node_feat, edge_feat, rij, W_radial, W_self, b_self, idx_i, idx_j):
    idx_i = idx_i.astype(jnp.int32)
    idx_j = idx_j.astype(jnp.int32)
    W2 = W_self * (1.0 / NORM_FACTOR)
    R, edge_out = _radial(rij, W_radial, edge_feat)
    partials = _edge_agg(node_feat, R, idx_i, idx_j)
    node_out = _update(partials, node_feat, W2, b_self)
    return (node_out, edge_out)


# speedup vs baseline: 1.0745x; 1.0745x over previous
import functools

import numpy as np
import jax
import jax.numpy as jnp
from jax import lax
from jax.experimental import pallas as pl
from jax.experimental.pallas import tpu as pltpu
from jax.experimental.pallas import tpu_sc as plsc

N = 10000
E = 320000
D = 128
NB = 8
CUTOFF = 5.0
GAMMA = 10.0
NORM_FACTOR = 32.0

NC = 2
NS = 16
NW = NC * NS
EPT = E // NW
K = 64
NCH = 156
KT = EPT - NCH * K
NPAIR = NCH // 2
NP = 10112
RPT = NP // NS

_CENTERS = [float(c) for c in np.linspace(0.0, CUTOFF, NB)]


_RW = 128
_RROWS = E // _RW
_BR = 20


def _pack2(x):
    u = lax.bitcast_convert_type(x, jnp.uint32) + jnp.uint32(0x8000)
    lo = lax.shift_right_logical(u[:, : D // 2], jnp.uint32(16))
    hi = u[:, D // 2 :] & jnp.uint32(0xFFFF0000)
    return lax.bitcast_convert_type(lo | hi, jnp.int32)


def _radial_body(rij_ref, wr_ref, out_ref):
    r = rij_ref[0]
    fc = 0.5 * (jnp.cos(np.pi / CUTOFF * r) + 1.0)
    fc = jnp.where(r < CUTOFF, fc, 0.0)
    lhs = jnp.concatenate(
        [jnp.exp(-GAMMA * (r - _CENTERS[b]) ** 2) * fc for b in range(NB)],
        axis=0)
    out_ref[...] = _pack2(lax.dot_general(
        lhs, wr_ref[...], (((0,), (0,)), ((), ())),
        preferred_element_type=jnp.float32))


def _radial(rij, W_radial):
    return pl.pallas_call(
        _radial_body,
        grid=(_RROWS // _BR,),
        in_specs=[
            pl.BlockSpec((1, 1, _BR * _RW), lambda i: (i, 0, 0)),
            pl.BlockSpec((NB, D), lambda i: (0, 0)),
        ],
        out_specs=pl.BlockSpec((_BR * _RW, D // 2), lambda i: (i, 0)),
        out_shape=jax.ShapeDtypeStruct((E, D // 2), jnp.int32),
    )(rij.reshape(_RROWS // _BR, 1, _BR * _RW), W_radial)


def _edge_copy_body(ef_ref, eo_ref):
    eo_ref[...] = ef_ref[...]


def _edge_copy(edge_feat):
    return pl.pallas_call(
        _edge_copy_body,
        grid=(_RROWS // _BR,),
        in_specs=[pl.BlockSpec((_BR * _RW, D), lambda i: (i, 0))],
        out_specs=pl.BlockSpec((_BR * _RW, D), lambda i: (i, 0)),
        out_shape=jax.ShapeDtypeStruct((E, D), jnp.float32),
    )(edge_feat)


_MESH = plsc.VectorSubcoreMesh(core_axis_name="c", subcore_axis_name="s")


@functools.partial(
    pl.kernel,
    out_type=jax.ShapeDtypeStruct((NC, NP, D), jnp.float32),
    mesh=_MESH,
    compiler_params=pltpu.CompilerParams(needs_layout_passes=False),
    scratch_types=[
        pltpu.VMEM((8 * K,), jnp.int32),
        pltpu.VMEM((8 * K,), jnp.int32),
        pltpu.VMEM((2, K, D // 2), jnp.int32),
        pltpu.VMEM((2, K, D), jnp.float32),
        pltpu.VMEM((2, K, D), jnp.float32),
        pltpu.VMEM_SHARED((NP, D), jnp.float32),
        pltpu.SemaphoreType.DMA,
        pltpu.SemaphoreType.DMA,
        pltpu.SemaphoreType.DMA,
        pltpu.SemaphoreType.DMA,
        pltpu.SemaphoreType.DMA,
        pltpu.SemaphoreType.DMA,
        pltpu.SemaphoreType.DMA,
        pltpu.SemaphoreType.DMA,
        pltpu.SemaphoreType.DMA,
        pltpu.SemaphoreType.DMA,
    ],
)
def _edge_agg(node_hbm, r_hbm, idxi_hbm, idxj_hbm, out_hbm,
              ij_v, ii_v, r_v, g_v, msg_v, agg,
              semr0, semr1, semg0, semg1, sems0, sems1,
              semij0, semij1, semii0, semii1):
    cid = lax.axis_index("c")
    sid = lax.axis_index("s")
    wid = cid * NS + sid
    ebase = wid * EPT
    semr = (semr0, semr1)
    semg = (semg0, semg1)
    sems = (sems0, sems1)
    semij = (semij0, semij1)
    semii = (semii0, semii1)

    def _zrow(k, _):
        for v in range(D // 16):
            msg_v[0, k, pl.ds(v * 16, 16)] = jnp.zeros((16,), jnp.float32)
        return 0

    lax.fori_loop(0, K, _zrow, 0)
    for z in range(RPT // K):
        pltpu.sync_copy(msg_v.at[0], agg.at[pl.ds(sid * RPT + z * K, K), :])
    _zrem = RPT - (RPT // K) * K
    if _zrem:
        pltpu.sync_copy(
            msg_v.at[0, pl.ds(0, _zrem), :],
            agg.at[pl.ds(sid * RPT + (RPT // K) * K, _zrem), :])
    plsc.subcore_barrier()

    def _issue_idx(p, off4, sem_par):
        pltpu.async_copy(idxj_hbm.at[pl.ds(ebase + p * 2 * K, 2 * K)],
                         ij_v.at[pl.ds(off4, 2 * K)], semij[sem_par])
        pltpu.async_copy(idxi_hbm.at[pl.ds(ebase + p * 2 * K, 2 * K)],
                         ii_v.at[pl.ds(off4, 2 * K)], semii[sem_par])

    def _wait_idx(p, off4, sem_par):
        pltpu.make_async_copy(idxj_hbm.at[pl.ds(ebase + p * 2 * K, 2 * K)],
                              ij_v.at[pl.ds(off4, 2 * K)],
                              semij[sem_par]).wait()
        pltpu.make_async_copy(idxi_hbm.at[pl.ds(ebase + p * 2 * K, 2 * K)],
                              ii_v.at[pl.ds(off4, 2 * K)],
                              semii[sem_par]).wait()

    def _issue_data(t, off4, c, slot):
        pltpu.async_copy(r_hbm.at[pl.ds(ebase + t * K, K), :],
                         r_v.at[slot], semr[slot])
        pltpu.async_copy(node_hbm.at[ij_v.at[pl.ds(off4 + c * K, K)]],
                         g_v.at[slot], semg[slot])

    def _mul(slot, rows):
        def _row(k, _):
            for v in range(D // 32):
                sl = pl.ds(v * 16, 16)
                sh = pl.ds(D // 2 + v * 16, 16)
                rw = r_v[slot, k, sl]
                ra = plsc.bitcast(lax.shift_left(rw, 16), jnp.float32)
                rb = plsc.bitcast(rw & jnp.int32(-65536), jnp.float32)
                msg_v[slot, k, sl] = ra * g_v[slot, k, sl]
                msg_v[slot, k, sh] = rb * g_v[slot, k, sh]
            return 0

        lax.fori_loop(0, rows, _row, 0)

    def _process(t, off4, c, slot, first):
        pltpu.make_async_copy(r_hbm.at[pl.ds(ebase + t * K, K), :],
                              r_v.at[slot], semr[slot]).wait()
        pltpu.make_async_copy(node_hbm.at[ij_v.at[pl.ds(off4 + c * K, K)]],
                              g_v.at[slot], semg[slot]).wait()

        @pl.when(jnp.logical_not(first))
        def _():
            pltpu.make_async_copy(
                msg_v.at[slot], agg.at[ii_v.at[pl.ds(off4 + c * K, K)]],
                sems[slot]).wait()

        _mul(slot, K)
        pltpu.async_copy(msg_v.at[slot],
                         agg.at[ii_v.at[pl.ds(off4 + c * K, K)]],
                         sems[slot], add=True)

    pltpu.sync_copy(idxj_hbm.at[pl.ds(ebase, 2 * K)],
                    ij_v.at[pl.ds(0, 2 * K)])
    pltpu.sync_copy(idxi_hbm.at[pl.ds(ebase, 2 * K)],
                    ii_v.at[pl.ds(0, 2 * K)])
    _issue_idx(1, 2 * K, 1)
    _issue_data(0, 0, 0, 0)
    _issue_data(1, 0, 1, 1)

    def _pair_body(u, par):
        off4 = (u % 4) * 2 * K
        off4n = ((u + 1) % 4) * 2 * K

        @pl.when(u + 2 < NPAIR)
        def _():
            _issue_idx(u + 2, ((u + 2) % 4) * 2 * K, par)

        _process(2 * u, off4, 0, 0, u == 0)

        @pl.when(u + 1 < NPAIR)
        def _():
            _wait_idx(u + 1, off4n, 1 - par)
            _issue_data(2 * u + 2, off4n, 0, 0)

        _process(2 * u + 1, off4, 1, 1, u == 0)

        @pl.when(u + 1 < NPAIR)
        def _():
            _issue_data(2 * u + 3, off4n, 1, 1)

    def _pairs(h, _):
        _pair_body(2 * h, 0)
        _pair_body(2 * h + 1, 1)
        return 0

    lax.fori_loop(0, NPAIR // 2, _pairs, 0)

    for slot in range(2):
        pltpu.make_async_copy(
            msg_v.at[slot], agg.at[ii_v.at[pl.ds(slot * K, K)]],
            sems[slot]).wait()

    tbase = ebase + NCH * K
    pltpu.sync_copy(idxj_hbm.at[pl.ds(tbase, KT)], ij_v.at[pl.ds(0, KT)])
    pltpu.sync_copy(idxi_hbm.at[pl.ds(tbase, KT)], ii_v.at[pl.ds(0, KT)])
    pltpu.sync_copy(r_hbm.at[pl.ds(tbase, KT), :], r_v.at[0, pl.ds(0, KT), :])
    pltpu.async_copy(node_hbm.at[ij_v.at[pl.ds(0, KT)]],
                     g_v.at[0, pl.ds(0, KT), :], semg[0]).wait()
    _mul(0, KT)
    pltpu.sync_copy(msg_v.at[0, pl.ds(0, KT), :],
                    agg.at[ii_v.at[pl.ds(0, KT)]], add=True)
    plsc.subcore_barrier()

    pltpu.sync_copy(agg.at[pl.ds(sid * RPT, RPT), :],
                    out_hbm.at[cid, pl.ds(sid * RPT, RPT), :])


_BN = 1000


def _update_body(p_ref, nf_ref, ws_ref, b_ref, o_ref):
    agg = p_ref[0] + p_ref[1]
    h = jnp.dot(agg, ws_ref[...], preferred_element_type=jnp.float32)
    h = h + b_ref[...]
    o_ref[...] = nf_ref[...] + h * jax.nn.sigmoid(h)


def _update(partials, node_feat, W_self, b_self):
    return pl.pallas_call(
        _update_body,
        grid=(N // _BN,),
        in_specs=[
            pl.BlockSpec((NC, _BN, D), lambda i: (0, i, 0)),
            pl.BlockSpec((_BN, D), lambda i: (i, 0)),
            pl.BlockSpec((D, D), lambda i: (0, 0)),
            pl.BlockSpec((1, D), lambda i: (0, 0)),
        ],
        out_specs=pl.BlockSpec((_BN, D), lambda i: (i, 0)),
        out_shape=jax.ShapeDtypeStruct((N, D), jnp.float32),
    )(partials, node_feat, W_self, b_self.reshape(1, D))


def kernel(node_feat, edge_feat, rij, W_radial, W_self, b_self, idx_i, idx_j):
    idx_i = idx_i.astype(jnp.int32)
    idx_j = idx_j.astype(jnp.int32)
    W2 = W_self * (1.0 / NORM_FACTOR)
    R = _radial(rij, W_radial)
    partials = _edge_agg(node_feat, R, idx_i, idx_j)
    edge_out = _edge_copy(edge_feat)
    node_out = _update(partials, node_feat, W2, b_self)
    return (node_out, edge_out)


# speedup vs baseline: 1.1356x; 1.0569x over previous
import functools

import numpy as np
import jax
import jax.numpy as jnp
from jax import lax
from jax.experimental import pallas as pl
from jax.experimental.pallas import tpu as pltpu
from jax.experimental.pallas import tpu_sc as plsc

N = 10000
E = 320000
D = 128
NB = 8
CUTOFF = 5.0
GAMMA = 10.0
NORM_FACTOR = 32.0

NC = 2
NS = 16
NW = NC * NS
EPT = E // NW
K = 64
NCH = 156
KT = EPT - NCH * K
NPAIR = NCH // 2
NP = 10112
RPT = NP // NS

_CENTERS = [float(c) for c in np.linspace(0.0, CUTOFF, NB)]


_RW = 128
_RROWS = E // _RW
_BR = 50


def _pack2(x):
    u = lax.bitcast_convert_type(x, jnp.uint32) + jnp.uint32(0x8000)
    lo = lax.shift_right_logical(u[:, : D // 2], jnp.uint32(16))
    hi = u[:, D // 2 :] & jnp.uint32(0xFFFF0000)
    return lax.bitcast_convert_type(lo | hi, jnp.int32)


def _radial_body(rij_ref, wr_ref, out_ref):
    r = rij_ref[0]
    fc = 0.5 * (jnp.cos(np.pi / CUTOFF * r) + 1.0)
    fc = jnp.where(r < CUTOFF, fc, 0.0)
    lhs = jnp.concatenate(
        [jnp.exp(-GAMMA * (r - _CENTERS[b]) ** 2) * fc for b in range(NB)],
        axis=0)
    out_ref[...] = _pack2(lax.dot_general(
        lhs, wr_ref[...], (((0,), (0,)), ((), ())),
        preferred_element_type=jnp.float32))


def _radial(rij, W_radial):
    return pl.pallas_call(
        _radial_body,
        grid=(_RROWS // _BR,),
        in_specs=[
            pl.BlockSpec((1, 1, _BR * _RW), lambda i: (i, 0, 0)),
            pl.BlockSpec((NB, D), lambda i: (0, 0)),
        ],
        out_specs=pl.BlockSpec((_BR * _RW, D // 2), lambda i: (i, 0)),
        out_shape=jax.ShapeDtypeStruct((E, D // 2), jnp.int32),
    )(rij.reshape(_RROWS // _BR, 1, _BR * _RW), W_radial)


def _edge_copy_body(ef_ref, eo_ref):
    eo_ref[...] = ef_ref[...]


def _edge_copy(edge_feat):
    return pl.pallas_call(
        _edge_copy_body,
        grid=(_RROWS // _BR,),
        in_specs=[pl.BlockSpec((_BR * _RW, D), lambda i: (i, 0))],
        out_specs=pl.BlockSpec((_BR * _RW, D), lambda i: (i, 0)),
        out_shape=jax.ShapeDtypeStruct((E, D), jnp.float32),
    )(edge_feat)


_MESH = plsc.VectorSubcoreMesh(core_axis_name="c", subcore_axis_name="s")


@functools.partial(
    pl.kernel,
    out_type=jax.ShapeDtypeStruct((NC, NP, D), jnp.float32),
    mesh=_MESH,
    compiler_params=pltpu.CompilerParams(needs_layout_passes=False),
    scratch_types=[
        pltpu.VMEM((8 * K,), jnp.int32),
        pltpu.VMEM((8 * K,), jnp.int32),
        pltpu.VMEM((2, K, D // 2), jnp.int32),
        pltpu.VMEM((2, K, D), jnp.float32),
        pltpu.VMEM((2, K, D), jnp.float32),
        pltpu.VMEM_SHARED((NP, D), jnp.float32),
        pltpu.SemaphoreType.DMA,
        pltpu.SemaphoreType.DMA,
        pltpu.SemaphoreType.DMA,
        pltpu.SemaphoreType.DMA,
        pltpu.SemaphoreType.DMA,
        pltpu.SemaphoreType.DMA,
        pltpu.SemaphoreType.DMA,
        pltpu.SemaphoreType.DMA,
        pltpu.SemaphoreType.DMA,
        pltpu.SemaphoreType.DMA,
    ],
)
def _edge_agg(node_hbm, r_hbm, idxi_hbm, idxj_hbm, out_hbm,
              ij_v, ii_v, r_v, g_v, msg_v, agg,
              semr0, semr1, semg0, semg1, sems0, sems1,
              semij0, semij1, semii0, semii1):
    cid = lax.axis_index("c")
    sid = lax.axis_index("s")
    wid = cid * NS + sid
    ebase = wid * EPT
    semr = (semr0, semr1)
    semg = (semg0, semg1)
    sems = (sems0, sems1)
    semij = (semij0, semij1)
    semii = (semii0, semii1)

    def _zrow(k, _):
        for v in range(D // 16):
            msg_v[0, k, pl.ds(v * 16, 16)] = jnp.zeros((16,), jnp.float32)
        return 0

    lax.fori_loop(0, K, _zrow, 0)
    for z in range(RPT // K):
        pltpu.sync_copy(msg_v.at[0], agg.at[pl.ds(sid * RPT + z * K, K), :])
    _zrem = RPT - (RPT // K) * K
    if _zrem:
        pltpu.sync_copy(
            msg_v.at[0, pl.ds(0, _zrem), :],
            agg.at[pl.ds(sid * RPT + (RPT // K) * K, _zrem), :])
    plsc.subcore_barrier()

    def _issue_idx(p, off4, sem_par):
        pltpu.async_copy(idxj_hbm.at[pl.ds(ebase + p * 2 * K, 2 * K)],
                         ij_v.at[pl.ds(off4, 2 * K)], semij[sem_par])
        pltpu.async_copy(idxi_hbm.at[pl.ds(ebase + p * 2 * K, 2 * K)],
                         ii_v.at[pl.ds(off4, 2 * K)], semii[sem_par])

    def _wait_idx(p, off4, sem_par):
        pltpu.make_async_copy(idxj_hbm.at[pl.ds(ebase + p * 2 * K, 2 * K)],
                              ij_v.at[pl.ds(off4, 2 * K)],
                              semij[sem_par]).wait()
        pltpu.make_async_copy(idxi_hbm.at[pl.ds(ebase + p * 2 * K, 2 * K)],
                              ii_v.at[pl.ds(off4, 2 * K)],
                              semii[sem_par]).wait()

    def _issue_data(t, off4, c, slot):
        pltpu.async_copy(r_hbm.at[pl.ds(ebase + t * K, K), :],
                         r_v.at[slot], semr[slot])
        pltpu.async_copy(node_hbm.at[ij_v.at[pl.ds(off4 + c * K, K)]],
                         g_v.at[slot], semg[slot])

    def _mul(slot, rows):
        def _row(k, _):
            for v in range(D // 32):
                sl = pl.ds(v * 16, 16)
                sh = pl.ds(D // 2 + v * 16, 16)
                rw = r_v[slot, k, sl]
                ra = plsc.bitcast(lax.shift_left(rw, 16), jnp.float32)
                rb = plsc.bitcast(rw & jnp.int32(-65536), jnp.float32)
                msg_v[slot, k, sl] = ra * g_v[slot, k, sl]
                msg_v[slot, k, sh] = rb * g_v[slot, k, sh]
            return 0

        lax.fori_loop(0, rows, _row, 0)

    def _process(t, off4, c, slot, first):
        pltpu.make_async_copy(r_hbm.at[pl.ds(ebase + t * K, K), :],
                              r_v.at[slot], semr[slot]).wait()
        pltpu.make_async_copy(node_hbm.at[ij_v.at[pl.ds(off4 + c * K, K)]],
                              g_v.at[slot], semg[slot]).wait()

        @pl.when(jnp.logical_not(first))
        def _():
            pltpu.make_async_copy(
                msg_v.at[slot], agg.at[ii_v.at[pl.ds(off4 + c * K, K)]],
                sems[slot]).wait()

        _mul(slot, K)
        pltpu.async_copy(msg_v.at[slot],
                         agg.at[ii_v.at[pl.ds(off4 + c * K, K)]],
                         sems[slot], add=True)

    pltpu.sync_copy(idxj_hbm.at[pl.ds(ebase, 2 * K)],
                    ij_v.at[pl.ds(0, 2 * K)])
    pltpu.sync_copy(idxi_hbm.at[pl.ds(ebase, 2 * K)],
                    ii_v.at[pl.ds(0, 2 * K)])
    _issue_idx(1, 2 * K, 1)
    _issue_data(0, 0, 0, 0)
    _issue_data(1, 0, 1, 1)

    def _pair_body(u, par):
        off4 = (u % 4) * 2 * K
        off4n = ((u + 1) % 4) * 2 * K

        @pl.when(u + 2 < NPAIR)
        def _():
            _issue_idx(u + 2, ((u + 2) % 4) * 2 * K, par)

        _process(2 * u, off4, 0, 0, u == 0)

        @pl.when(u + 1 < NPAIR)
        def _():
            _wait_idx(u + 1, off4n, 1 - par)
            _issue_data(2 * u + 2, off4n, 0, 0)

        _process(2 * u + 1, off4, 1, 1, u == 0)

        @pl.when(u + 1 < NPAIR)
        def _():
            _issue_data(2 * u + 3, off4n, 1, 1)

    def _pairs(h, _):
        _pair_body(2 * h, 0)
        _pair_body(2 * h + 1, 1)
        return 0

    lax.fori_loop(0, NPAIR // 2, _pairs, 0)

    for slot in range(2):
        pltpu.make_async_copy(
            msg_v.at[slot], agg.at[ii_v.at[pl.ds(slot * K, K)]],
            sems[slot]).wait()

    tbase = ebase + NCH * K
    pltpu.sync_copy(idxj_hbm.at[pl.ds(tbase, KT)], ij_v.at[pl.ds(0, KT)])
    pltpu.sync_copy(idxi_hbm.at[pl.ds(tbase, KT)], ii_v.at[pl.ds(0, KT)])
    pltpu.sync_copy(r_hbm.at[pl.ds(tbase, KT), :], r_v.at[0, pl.ds(0, KT), :])
    pltpu.async_copy(node_hbm.at[ij_v.at[pl.ds(0, KT)]],
                     g_v.at[0, pl.ds(0, KT), :], semg[0]).wait()
    _mul(0, KT)
    pltpu.sync_copy(msg_v.at[0, pl.ds(0, KT), :],
                    agg.at[ii_v.at[pl.ds(0, KT)]], add=True)
    plsc.subcore_barrier()

    pltpu.sync_copy(agg.at[pl.ds(sid * RPT, RPT), :],
                    out_hbm.at[cid, pl.ds(sid * RPT, RPT), :])


_BN = 1000


def _update_body(p_ref, nf_ref, ws_ref, b_ref, o_ref):
    agg = p_ref[0] + p_ref[1]
    h = jnp.dot(agg, ws_ref[...], preferred_element_type=jnp.float32)
    h = h + b_ref[...]
    o_ref[...] = nf_ref[...] + h * jax.nn.sigmoid(h)


def _update(partials, node_feat, W_self, b_self):
    return pl.pallas_call(
        _update_body,
        grid=(N // _BN,),
        in_specs=[
            pl.BlockSpec((NC, _BN, D), lambda i: (0, i, 0)),
            pl.BlockSpec((_BN, D), lambda i: (i, 0)),
            pl.BlockSpec((D, D), lambda i: (0, 0)),
            pl.BlockSpec((1, D), lambda i: (0, 0)),
        ],
        out_specs=pl.BlockSpec((_BN, D), lambda i: (i, 0)),
        out_shape=jax.ShapeDtypeStruct((N, D), jnp.float32),
    )(partials, node_feat, W_self, b_self.reshape(1, D))


def kernel(node_feat, edge_feat, rij, W_radial, W_self, b_self, idx_i, idx_j):
    idx_i = idx_i.astype(jnp.int32)
    idx_j = idx_j.astype(jnp.int32)
    W2 = W_self * (1.0 / NORM_FACTOR)
    R = _radial(rij, W_radial)
    partials = _edge_agg(node_feat, R, idx_i, idx_j)
    edge_out = _edge_copy(edge_feat)
    node_out = _update(partials, node_feat, W2, b_self)
    return (node_out, edge_out)


# speedup vs baseline: 1.1590x; 1.0206x over previous
import functools

import numpy as np
import jax
import jax.numpy as jnp
from jax import lax
from jax.experimental import pallas as pl
from jax.experimental.pallas import tpu as pltpu
from jax.experimental.pallas import tpu_sc as plsc

N = 10000
E = 320000
D = 128
NB = 8
CUTOFF = 5.0
GAMMA = 10.0
NORM_FACTOR = 32.0

NC = 2
NS = 16
NW = NC * NS
EPT = E // NW
K = 64
NCH = 156
KT = EPT - NCH * K
NPAIR = NCH // 2
NP = 10112
RPT = NP // NS

_CENTERS = [float(c) for c in np.linspace(0.0, CUTOFF, NB)]


_RW = 128
_RROWS = E // _RW
_BR = 100


def _pack2(x):
    u = lax.bitcast_convert_type(x, jnp.uint32) + jnp.uint32(0x8000)
    lo = lax.shift_right_logical(u[:, : D // 2], jnp.uint32(16))
    hi = u[:, D // 2 :] & jnp.uint32(0xFFFF0000)
    return lax.bitcast_convert_type(lo | hi, jnp.int32)


def _radial_body(rij_ref, wr_ref, out_ref):
    r = rij_ref[0]
    fc = 0.5 * (jnp.cos(np.pi / CUTOFF * r) + 1.0)
    fc = jnp.where(r < CUTOFF, fc, 0.0)
    lhs = jnp.concatenate(
        [jnp.exp(-GAMMA * (r - _CENTERS[b]) ** 2) * fc for b in range(NB)],
        axis=0)
    out_ref[...] = _pack2(lax.dot_general(
        lhs, wr_ref[...], (((0,), (0,)), ((), ())),
        preferred_element_type=jnp.float32))


def _radial(rij, W_radial):
    return pl.pallas_call(
        _radial_body,
        grid=(_RROWS // _BR,),
        in_specs=[
            pl.BlockSpec((1, 1, _BR * _RW), lambda i: (i, 0, 0)),
            pl.BlockSpec((NB, D), lambda i: (0, 0)),
        ],
        out_specs=pl.BlockSpec((_BR * _RW, D // 2), lambda i: (i, 0)),
        out_shape=jax.ShapeDtypeStruct((E, D // 2), jnp.int32),
    )(rij.reshape(_RROWS // _BR, 1, _BR * _RW), W_radial)


def _edge_copy_body(ef_ref, eo_ref):
    eo_ref[...] = ef_ref[...]


def _edge_copy(edge_feat):
    return pl.pallas_call(
        _edge_copy_body,
        grid=(_RROWS // _BR,),
        in_specs=[pl.BlockSpec((_BR * _RW, D), lambda i: (i, 0))],
        out_specs=pl.BlockSpec((_BR * _RW, D), lambda i: (i, 0)),
        out_shape=jax.ShapeDtypeStruct((E, D), jnp.float32),
    )(edge_feat)


_MESH = plsc.VectorSubcoreMesh(core_axis_name="c", subcore_axis_name="s")


@functools.partial(
    pl.kernel,
    out_type=jax.ShapeDtypeStruct((NC, NP, D), jnp.float32),
    mesh=_MESH,
    compiler_params=pltpu.CompilerParams(needs_layout_passes=False),
    scratch_types=[
        pltpu.VMEM((8 * K,), jnp.int32),
        pltpu.VMEM((8 * K,), jnp.int32),
        pltpu.VMEM((2, K, D // 2), jnp.int32),
        pltpu.VMEM((2, K, D), jnp.float32),
        pltpu.VMEM((2, K, D), jnp.float32),
        pltpu.VMEM_SHARED((NP, D), jnp.float32),
        pltpu.SemaphoreType.DMA,
        pltpu.SemaphoreType.DMA,
        pltpu.SemaphoreType.DMA,
        pltpu.SemaphoreType.DMA,
        pltpu.SemaphoreType.DMA,
        pltpu.SemaphoreType.DMA,
        pltpu.SemaphoreType.DMA,
        pltpu.SemaphoreType.DMA,
        pltpu.SemaphoreType.DMA,
        pltpu.SemaphoreType.DMA,
    ],
)
def _edge_agg(node_hbm, r_hbm, idxi_hbm, idxj_hbm, out_hbm,
              ij_v, ii_v, r_v, g_v, msg_v, agg,
              semr0, semr1, semg0, semg1, sems0, sems1,
              semij0, semij1, semii0, semii1):
    cid = lax.axis_index("c")
    sid = lax.axis_index("s")
    wid = cid * NS + sid
    ebase = wid * EPT
    semr = (semr0, semr1)
    semg = (semg0, semg1)
    sems = (sems0, sems1)
    semij = (semij0, semij1)
    semii = (semii0, semii1)

    def _zrow(k, _):
        for v in range(D // 16):
            msg_v[0, k, pl.ds(v * 16, 16)] = jnp.zeros((16,), jnp.float32)
        return 0

    lax.fori_loop(0, K, _zrow, 0)
    for z in range(RPT // K):
        pltpu.sync_copy(msg_v.at[0], agg.at[pl.ds(sid * RPT + z * K, K), :])
    _zrem = RPT - (RPT // K) * K
    if _zrem:
        pltpu.sync_copy(
            msg_v.at[0, pl.ds(0, _zrem), :],
            agg.at[pl.ds(sid * RPT + (RPT // K) * K, _zrem), :])
    plsc.subcore_barrier()

    def _issue_idx(p, off4, sem_par):
        pltpu.async_copy(idxj_hbm.at[pl.ds(ebase + p * 2 * K, 2 * K)],
                         ij_v.at[pl.ds(off4, 2 * K)], semij[sem_par])
        pltpu.async_copy(idxi_hbm.at[pl.ds(ebase + p * 2 * K, 2 * K)],
                         ii_v.at[pl.ds(off4, 2 * K)], semii[sem_par])

    def _wait_idx(p, off4, sem_par):
        pltpu.make_async_copy(idxj_hbm.at[pl.ds(ebase + p * 2 * K, 2 * K)],
                              ij_v.at[pl.ds(off4, 2 * K)],
                              semij[sem_par]).wait()
        pltpu.make_async_copy(idxi_hbm.at[pl.ds(ebase + p * 2 * K, 2 * K)],
                              ii_v.at[pl.ds(off4, 2 * K)],
                              semii[sem_par]).wait()

    def _issue_data(t, off4, c, slot):
        pltpu.async_copy(r_hbm.at[pl.ds(ebase + t * K, K), :],
                         r_v.at[slot], semr[slot])
        pltpu.async_copy(node_hbm.at[ij_v.at[pl.ds(off4 + c * K, K)]],
                         g_v.at[slot], semg[slot])

    def _mul(slot, rows):
        def _row(k, _):
            for v in range(D // 32):
                sl = pl.ds(v * 16, 16)
                sh = pl.ds(D // 2 + v * 16, 16)
                rw = r_v[slot, k, sl]
                ra = plsc.bitcast(lax.shift_left(rw, 16), jnp.float32)
                rb = plsc.bitcast(rw & jnp.int32(-65536), jnp.float32)
                msg_v[slot, k, sl] = ra * g_v[slot, k, sl]
                msg_v[slot, k, sh] = rb * g_v[slot, k, sh]
            return 0

        lax.fori_loop(0, rows, _row, 0)

    def _process(t, off4, c, slot, first):
        pltpu.make_async_copy(r_hbm.at[pl.ds(ebase + t * K, K), :],
                              r_v.at[slot], semr[slot]).wait()
        pltpu.make_async_copy(node_hbm.at[ij_v.at[pl.ds(off4 + c * K, K)]],
                              g_v.at[slot], semg[slot]).wait()

        @pl.when(jnp.logical_not(first))
        def _():
            pltpu.make_async_copy(
                msg_v.at[slot], agg.at[ii_v.at[pl.ds(off4 + c * K, K)]],
                sems[slot]).wait()

        _mul(slot, K)
        pltpu.async_copy(msg_v.at[slot],
                         agg.at[ii_v.at[pl.ds(off4 + c * K, K)]],
                         sems[slot], add=True)

    pltpu.sync_copy(idxj_hbm.at[pl.ds(ebase, 2 * K)],
                    ij_v.at[pl.ds(0, 2 * K)])
    pltpu.sync_copy(idxi_hbm.at[pl.ds(ebase, 2 * K)],
                    ii_v.at[pl.ds(0, 2 * K)])
    _issue_idx(1, 2 * K, 1)
    _issue_data(0, 0, 0, 0)
    _issue_data(1, 0, 1, 1)

    def _pair_body(u, par):
        off4 = (u % 4) * 2 * K
        off4n = ((u + 1) % 4) * 2 * K

        @pl.when(u + 2 < NPAIR)
        def _():
            _issue_idx(u + 2, ((u + 2) % 4) * 2 * K, par)

        _process(2 * u, off4, 0, 0, u == 0)

        @pl.when(u + 1 < NPAIR)
        def _():
            _wait_idx(u + 1, off4n, 1 - par)
            _issue_data(2 * u + 2, off4n, 0, 0)

        _process(2 * u + 1, off4, 1, 1, u == 0)

        @pl.when(u + 1 < NPAIR)
        def _():
            _issue_data(2 * u + 3, off4n, 1, 1)

    def _pairs(h, _):
        _pair_body(2 * h, 0)
        _pair_body(2 * h + 1, 1)
        return 0

    lax.fori_loop(0, NPAIR // 2, _pairs, 0)

    for slot in range(2):
        pltpu.make_async_copy(
            msg_v.at[slot], agg.at[ii_v.at[pl.ds(slot * K, K)]],
            sems[slot]).wait()

    tbase = ebase + NCH * K
    pltpu.sync_copy(idxj_hbm.at[pl.ds(tbase, KT)], ij_v.at[pl.ds(0, KT)])
    pltpu.sync_copy(idxi_hbm.at[pl.ds(tbase, KT)], ii_v.at[pl.ds(0, KT)])
    pltpu.sync_copy(r_hbm.at[pl.ds(tbase, KT), :], r_v.at[0, pl.ds(0, KT), :])
    pltpu.async_copy(node_hbm.at[ij_v.at[pl.ds(0, KT)]],
                     g_v.at[0, pl.ds(0, KT), :], semg[0]).wait()
    _mul(0, KT)
    pltpu.sync_copy(msg_v.at[0, pl.ds(0, KT), :],
                    agg.at[ii_v.at[pl.ds(0, KT)]], add=True)
    plsc.subcore_barrier()

    pltpu.sync_copy(agg.at[pl.ds(sid * RPT, RPT), :],
                    out_hbm.at[cid, pl.ds(sid * RPT, RPT), :])


_BN = 1000


def _update_body(p_ref, nf_ref, ws_ref, b_ref, o_ref):
    agg = p_ref[0] + p_ref[1]
    h = jnp.dot(agg, ws_ref[...], preferred_element_type=jnp.float32)
    h = h + b_ref[...]
    o_ref[...] = nf_ref[...] + h * jax.nn.sigmoid(h)


def _update(partials, node_feat, W_self, b_self):
    return pl.pallas_call(
        _update_body,
        grid=(N // _BN,),
        in_specs=[
            pl.BlockSpec((NC, _BN, D), lambda i: (0, i, 0)),
            pl.BlockSpec((_BN, D), lambda i: (i, 0)),
            pl.BlockSpec((D, D), lambda i: (0, 0)),
            pl.BlockSpec((1, D), lambda i: (0, 0)),
        ],
        out_specs=pl.BlockSpec((_BN, D), lambda i: (i, 0)),
        out_shape=jax.ShapeDtypeStruct((N, D), jnp.float32),
    )(partials, node_feat, W_self, b_self.reshape(1, D))


def kernel(node_feat, edge_feat, rij, W_radial, W_self, b_self, idx_i, idx_j):
    idx_i = idx_i.astype(jnp.int32)
    idx_j = idx_j.astype(jnp.int32)
    W2 = W_self * (1.0 / NORM_FACTOR)
    R = _radial(rij, W_radial)
    partials = _edge_agg(node_feat, R, idx_i, idx_j)
    edge_out = _edge_copy(edge_feat)
    node_out = _update(partials, node_feat, W2, b_self)
    return (node_out, edge_out)
